# Initial kernel scaffold; baseline (speedup 1.0000x reference)
import jax
import jax.numpy as jnp
from jax.experimental import pallas as pl


def kernel(x):
    raise NotImplementedError("write your pallas kernel here")


# speedup vs baseline: 4.7249x; 4.7249x over previous
import functools

import numpy as np

import jax
import jax.numpy as jnp
from jax import lax
from jax.experimental import pallas as pl
from jax.experimental.pallas import tpu as pltpu
from jax.experimental.pallas import tpu_sc as plsc

_B = 64
_N = 100000
_W = 4000
_NWIN = _N // _W
_VPW = _W // 16
_NBINS = 256
_NPASS = 4
_NC = 2
_NS = 16
_ROWS_PER_TILE = _B // (_NC * _NS)

_MSB = np.uint32(0x80000000)
_ALL1 = np.uint32(0xFFFFFFFF)


def _to_sortable(u):
    return jnp.where((u >> 31) == 1, u ^ _ALL1, u | _MSB)


def _from_sortable(k):
    return jnp.where((k >> 31) == 1, k ^ _MSB, k ^ _ALL1)


def _sc_sort_body(x_hbm, out_hbm, tmp_hbm, win, dst, hist, offs, owin):
    cid = lax.axis_index("c")
    sid = lax.axis_index("s")
    wid = sid * _NC + cid
    zeros16 = jnp.zeros((16,), jnp.int32)

    for r in range(_ROWS_PER_TILE):
        row = wid * _ROWS_PER_TILE + r

        def zero_body(i, c):
            hist[pl.ds(i * 16, 16)] = zeros16
            return c

        lax.fori_loop(0, (_NPASS * _NBINS) // 16, zero_body, 0)

        def hist_win(w, c):
            pltpu.sync_copy(x_hbm.at[pl.ds(row * _N + w * _W, _W)], win)

            def hist_vec(v, c2):
                raw = win[pl.ds(v * 16, 16)]
                key = _to_sortable(plsc.bitcast(raw, jnp.uint32))
                for p in range(_NPASS):
                    dig = plsc.bitcast(
                        (key >> (8 * p)) & 0xFF, jnp.int32)
                    cnt, last = plsc.scan_count(dig)
                    plsc.addupdate_scatter(
                        hist, [dig + (p * _NBINS)], cnt, mask=last)
                return c2

            return lax.fori_loop(0, _VPW, hist_vec, c)

        lax.fori_loop(0, _NWIN, hist_win, 0)

        for p in range(_NPASS):
            def scan_body(i, carry):
                h = hist[pl.ds(p * _NBINS + i * 16, 16)]
                incl = plsc.cumsum(h)
                offs[pl.ds(i * 16, 16)] = incl - h + carry
                return carry + jnp.sum(h)

            lax.fori_loop(0, _NBINS // 16, scan_body, jnp.int32(0))

            src = x_hbm if p == 0 else tmp_hbm

            def perm_win(w, c):
                pltpu.sync_copy(src.at[pl.ds(row * _N + w * _W, _W)], win)

                def perm_vec(v, c2):
                    raw = win[pl.ds(v * 16, 16)]
                    ku = plsc.bitcast(raw, jnp.uint32)
                    if p == 0:
                        ku = _to_sortable(ku)
                    dig = plsc.bitcast(
                        (ku >> (8 * p)) & 0xFF, jnp.int32)
                    cnt, last = plsc.scan_count(dig)
                    base = plsc.load_gather(offs, [dig])
                    plsc.store_scatter(
                        dst, [base + cnt - 1], plsc.bitcast(ku, jnp.int32))
                    plsc.addupdate_scatter(offs, [dig], cnt, mask=last)
                    return c2

                return lax.fori_loop(0, _VPW, perm_vec, c)

            lax.fori_loop(0, _NWIN, perm_win, 0)

            if p < _NPASS - 1:
                pltpu.sync_copy(dst, tmp_hbm.at[pl.ds(row * _N, _N)])
            else:
                def out_win(w, c):
                    def out_vec(v, c2):
                        kk = plsc.bitcast(
                            dst[pl.ds(w * _W + v * 16, 16)], jnp.uint32)
                        owin[pl.ds(v * 16, 16)] = plsc.bitcast(
                            _from_sortable(kk), jnp.float32)
                        return c2

                    lax.fori_loop(0, _VPW, out_vec, c)
                    pltpu.sync_copy(owin, out_hbm.at[pl.ds(row * _N + w * _W, _W)])
                    return c

                lax.fori_loop(0, _NWIN, out_win, 0)


_sc_sort = functools.partial(
    pl.kernel,
    out_type=(
        jax.ShapeDtypeStruct((_B * _N,), jnp.float32),
        jax.ShapeDtypeStruct((_B * _N,), jnp.int32),
    ),
    mesh=plsc.VectorSubcoreMesh(
        core_axis_name="c", subcore_axis_name="s",
        num_cores=_NC, num_subcores=_NS),
    compiler_params=pltpu.CompilerParams(needs_layout_passes=False),
    scratch_types=[
        pltpu.VMEM((_W,), jnp.int32),
        pltpu.VMEM((_N,), jnp.int32),
        pltpu.VMEM((_NPASS * _NBINS,), jnp.int32),
        pltpu.VMEM((_NBINS,), jnp.int32),
        pltpu.VMEM((_W,), jnp.float32),
    ],
)(_sc_sort_body)


_LDJ_R, _LDJ_C = 8, 12544


def _ldj_body(o_ref):
    i0 = lax.broadcasted_iota(jnp.int32, (_LDJ_R, _LDJ_C), 0)
    i1 = lax.broadcasted_iota(jnp.int32, (_LDJ_R, _LDJ_C), 1)
    flat = i0 * _LDJ_C + i1
    val = jnp.log((flat + 1).astype(jnp.float32))
    s = jnp.sum(jnp.where(flat < _N, val, 0.0))
    o_ref[...] = jnp.full((_B, 1), -s, jnp.float32)


_ldj_call = pl.pallas_call(
    _ldj_body,
    out_shape=jax.ShapeDtypeStruct((_B, 1), jnp.float32),
)


def kernel(x):
    xb = lax.bitcast_convert_type(x, jnp.int32).reshape(_B * _N)
    z, _ = _sc_sort(xb)
    ldj = _ldj_call().reshape(_B)
    return (z.reshape(_B, _N), ldj)


# speedup vs baseline: 6.8791x; 1.4559x over previous
import functools

import numpy as np

import jax
import jax.numpy as jnp
from jax import lax
from jax.experimental import pallas as pl
from jax.experimental.pallas import tpu as pltpu
from jax.experimental.pallas import tpu_sc as plsc

_B = 64
_N = 100000
_W = 4000
_NWIN = _N // _W
_VPW = _W // 16
_U = 5
_NC = 2
_NS = 16
_ROWS_PER_TILE = _B // (_NC * _NS)

_PASSES = ((0, 2048), (11, 2048), (22, 1024))
_HIST_BASE = (0, 2048, 4096)
_HIST_SIZE = 5120
_OFFS_SIZE = 2048

_MSB = np.uint32(0x80000000)
_ALL1 = np.uint32(0xFFFFFFFF)


def _to_sortable(u):
    return jnp.where((u >> 31) == 1, u ^ _ALL1, u | _MSB)


def _from_sortable(k):
    return jnp.where((k >> 31) == 1, k ^ _MSB, k ^ _ALL1)


def _sc_sort_body(x_hbm, out_hbm, tmp_hbm, win_a, win_b, dst, hist, offs,
                  sem_a, sem_b, sem_t):
    cid = lax.axis_index("c")
    sid = lax.axis_index("s")
    wid = sid * _NC + cid
    zeros16 = jnp.zeros((16,), jnp.int32)

    def sweep(src, row_base, compute):
        def start(buf, sem, w):
            pltpu.async_copy(src.at[pl.ds(row_base + w * _W, _W)], buf, sem)

        def wait(buf, sem):
            pltpu.make_async_copy(
                src.at[pl.ds(row_base, _W)], buf, sem).wait()

        start(win_a, sem_a, 0)

        def pair(i, c):
            w0 = 2 * i
            wait(win_a, sem_a)
            start(win_b, sem_b, w0 + 1)
            compute(win_a, w0)
            wait(win_b, sem_b)
            start(win_a, sem_a, w0 + 2)
            compute(win_b, w0 + 1)
            return c

        lax.fori_loop(0, (_NWIN - 1) // 2, pair, 0)
        wait(win_a, sem_a)
        compute(win_a, _NWIN - 1)

    for r in range(_ROWS_PER_TILE):
        row = wid * _ROWS_PER_TILE + r
        row_base = row * _N

        def zero_body(i, c):
            hist[pl.ds(i * 16, 16)] = zeros16
            return c

        lax.fori_loop(0, _HIST_SIZE // 16, zero_body, 0)

        def hist_compute(buf, w):
            def body(g, c):
                for u in range(_U):
                    raw = buf[pl.ds((g * _U + u) * 16, 16)]
                    key = _to_sortable(plsc.bitcast(raw, jnp.uint32))
                    for (shift, bins), hb in zip(_PASSES, _HIST_BASE):
                        dig = plsc.bitcast(
                            (key >> shift) & (bins - 1), jnp.int32)
                        cnt, last = plsc.scan_count(dig)
                        plsc.addupdate_scatter(
                            hist, [dig + hb], cnt, mask=last)
                return c

            lax.fori_loop(0, _VPW // _U, body, 0)

        sweep(x_hbm, row_base, hist_compute)

        for p, ((shift, bins), hb) in enumerate(zip(_PASSES, _HIST_BASE)):
            def scan_body(i, carry):
                h = hist[pl.ds(hb + i * 16, 16)]
                incl = plsc.cumsum(h)
                offs[pl.ds(i * 16, 16)] = incl - h + carry
                return carry + jnp.sum(h)

            lax.fori_loop(0, bins // 16, scan_body, jnp.int32(-1))

            if p > 0:
                pltpu.make_async_copy(
                    dst, tmp_hbm.at[pl.ds(row_base, _N)], sem_t).wait()

            last_pass = p == len(_PASSES) - 1

            def perm_compute(buf, w):
                def body(g, c):
                    for u in range(_U):
                        raw = buf[pl.ds((g * _U + u) * 16, 16)]
                        ku = plsc.bitcast(raw, jnp.uint32)
                        if p == 0:
                            ku = _to_sortable(ku)
                        dig = plsc.bitcast(
                            (ku >> shift) & (bins - 1), jnp.int32)
                        cnt, last = plsc.scan_count(dig)
                        base = plsc.load_gather(offs, [dig])
                        val = _from_sortable(ku) if last_pass else ku
                        plsc.store_scatter(
                            dst, [base + cnt], plsc.bitcast(val, jnp.int32))
                        plsc.addupdate_scatter(offs, [dig], cnt, mask=last)
                    return c

                lax.fori_loop(0, _VPW // _U, body, 0)

            src = x_hbm if p == 0 else tmp_hbm
            sweep(src, row_base, perm_compute)

            dst_hbm = out_hbm if last_pass else tmp_hbm
            pltpu.async_copy(dst, dst_hbm.at[pl.ds(row_base, _N)], sem_t)
            if last_pass:
                pltpu.make_async_copy(
                    dst, dst_hbm.at[pl.ds(row_base, _N)], sem_t).wait()


_sc_sort = functools.partial(
    pl.kernel,
    out_type=(
        jax.ShapeDtypeStruct((_B * _N,), jnp.int32),
        jax.ShapeDtypeStruct((_B * _N,), jnp.int32),
    ),
    mesh=plsc.VectorSubcoreMesh(
        core_axis_name="c", subcore_axis_name="s",
        num_cores=_NC, num_subcores=_NS),
    compiler_params=pltpu.CompilerParams(needs_layout_passes=False),
    scratch_types=[
        pltpu.VMEM((_W,), jnp.int32),
        pltpu.VMEM((_W,), jnp.int32),
        pltpu.VMEM((_N,), jnp.int32),
        pltpu.VMEM((_HIST_SIZE,), jnp.int32),
        pltpu.VMEM((_OFFS_SIZE,), jnp.int32),
        pltpu.SemaphoreType.DMA,
        pltpu.SemaphoreType.DMA,
        pltpu.SemaphoreType.DMA,
    ],
)(_sc_sort_body)


_LDJ_R, _LDJ_C = 8, 12544


def _ldj_body(o_ref):
    i0 = lax.broadcasted_iota(jnp.int32, (_LDJ_R, _LDJ_C), 0)
    i1 = lax.broadcasted_iota(jnp.int32, (_LDJ_R, _LDJ_C), 1)
    flat = i0 * _LDJ_C + i1
    val = jnp.log((flat + 1).astype(jnp.float32))
    s = jnp.sum(jnp.where(flat < _N, val, 0.0))
    o_ref[...] = jnp.full((_B, 1), -s, jnp.float32)


_ldj_call = pl.pallas_call(
    _ldj_body,
    out_shape=jax.ShapeDtypeStruct((_B, 1), jnp.float32),
)


def kernel(x):
    xb = lax.bitcast_convert_type(x, jnp.int32).reshape(_B * _N)
    z, _ = _sc_sort(xb)
    ldj = _ldj_call().reshape(_B)
    z = lax.bitcast_convert_type(z, jnp.float32).reshape(_B, _N)
    return (z, ldj)


# speedup vs baseline: 7.5788x; 1.1017x over previous
import functools

import numpy as np

import jax
import jax.numpy as jnp
from jax import lax
from jax.experimental import pallas as pl
from jax.experimental.pallas import tpu as pltpu
from jax.experimental.pallas import tpu_sc as plsc

_B = 64
_N = 100000
_W = 4000
_NWIN = _N // _W
_VPW = _W // 16
_U = 5
_NC = 2
_NS = 16
_ROWS_PER_TILE = _B // (_NC * _NS)

_PASSES = ((0, 2048), (11, 2048), (22, 1024))
_HIST_BASE = (0, 2048, 4096)
_HIST_SIZE = 5120
_OFFS_SIZE = 2048

_MSB = np.uint32(0x80000000)
_ALL1 = np.uint32(0xFFFFFFFF)


def _to_sortable(u):
    return jnp.where((u >> 31) == 1, u ^ _ALL1, u | _MSB)


def _from_sortable(k):
    return jnp.where((k >> 31) == 1, k ^ _MSB, k ^ _ALL1)


def _sc_sort_body(x_hbm, out_hbm, tmp_hbm, win_a, win_b, dst, hist, offs,
                  sem_a, sem_b, sem_t):
    cid = lax.axis_index("c")
    sid = lax.axis_index("s")
    wid = sid * _NC + cid
    zeros16 = jnp.zeros((16,), jnp.int32)
    ones16 = jnp.ones((16,), jnp.int32)

    def sweep(src, row_base, compute):
        def start(buf, sem, w):
            pltpu.async_copy(src.at[pl.ds(row_base + w * _W, _W)], buf, sem)

        def wait(buf, sem):
            pltpu.make_async_copy(
                src.at[pl.ds(row_base, _W)], buf, sem).wait()

        start(win_a, sem_a, 0)

        def pair(i, c):
            w0 = 2 * i
            wait(win_a, sem_a)
            start(win_b, sem_b, w0 + 1)
            compute(win_a, w0)
            wait(win_b, sem_b)
            start(win_a, sem_a, w0 + 2)
            compute(win_b, w0 + 1)
            return c

        lax.fori_loop(0, (_NWIN - 1) // 2, pair, 0)
        wait(win_a, sem_a)
        compute(win_a, _NWIN - 1)

    for r in range(_ROWS_PER_TILE):
        row = wid * _ROWS_PER_TILE + r
        row_base = row * _N

        def zero_body(i, c):
            hist[pl.ds(i * 16, 16)] = zeros16
            return c

        lax.fori_loop(0, _HIST_SIZE // 16, zero_body, 0)

        def hist_compute(buf, w):
            def body(g, c):
                for u in range(_U):
                    raw = buf[pl.ds((g * _U + u) * 16, 16)]
                    key = _to_sortable(plsc.bitcast(raw, jnp.uint32))
                    for (shift, bins), hb in zip(_PASSES, _HIST_BASE):
                        dig = plsc.bitcast(
                            (key >> shift) & (bins - 1), jnp.int32)
                        plsc.addupdate_scatter(hist, [dig + hb], ones16)
                return c

            lax.fori_loop(0, _VPW // _U, body, 0)

        sweep(x_hbm, row_base, hist_compute)

        for p, ((shift, bins), hb) in enumerate(zip(_PASSES, _HIST_BASE)):
            def scan_body(i, carry):
                h = hist[pl.ds(hb + i * 16, 16)]
                incl = plsc.cumsum(h)
                offs[pl.ds(i * 16, 16)] = incl - h + carry
                return carry + jnp.sum(h)

            lax.fori_loop(0, bins // 16, scan_body, jnp.int32(-1))

            if p > 0:
                pltpu.make_async_copy(
                    dst, tmp_hbm.at[pl.ds(row_base, _N)], sem_t).wait()

            last_pass = p == len(_PASSES) - 1

            def perm_compute(buf, w):
                def body(g, c):
                    for u in range(_U):
                        raw = buf[pl.ds((g * _U + u) * 16, 16)]
                        ku = plsc.bitcast(raw, jnp.uint32)
                        if p == 0:
                            ku = _to_sortable(ku)
                        dig = plsc.bitcast(
                            (ku >> shift) & (bins - 1), jnp.int32)
                        cnt, last = plsc.scan_count(dig)
                        base = plsc.load_gather(offs, [dig])
                        val = _from_sortable(ku) if last_pass else ku
                        plsc.store_scatter(
                            dst, [base + cnt], plsc.bitcast(val, jnp.int32))
                        plsc.addupdate_scatter(offs, [dig], cnt, mask=last)
                    return c

                lax.fori_loop(0, _VPW // _U, body, 0)

            src = x_hbm if p == 0 else tmp_hbm
            sweep(src, row_base, perm_compute)

            dst_hbm = out_hbm if last_pass else tmp_hbm
            pltpu.async_copy(dst, dst_hbm.at[pl.ds(row_base, _N)], sem_t)
            if last_pass:
                pltpu.make_async_copy(
                    dst, dst_hbm.at[pl.ds(row_base, _N)], sem_t).wait()


_sc_sort = functools.partial(
    pl.kernel,
    out_type=(
        jax.ShapeDtypeStruct((_B * _N,), jnp.int32),
        jax.ShapeDtypeStruct((_B * _N,), jnp.int32),
    ),
    mesh=plsc.VectorSubcoreMesh(
        core_axis_name="c", subcore_axis_name="s",
        num_cores=_NC, num_subcores=_NS),
    compiler_params=pltpu.CompilerParams(needs_layout_passes=False),
    scratch_types=[
        pltpu.VMEM((_W,), jnp.int32),
        pltpu.VMEM((_W,), jnp.int32),
        pltpu.VMEM((_N,), jnp.int32),
        pltpu.VMEM((_HIST_SIZE,), jnp.int32),
        pltpu.VMEM((_OFFS_SIZE,), jnp.int32),
        pltpu.SemaphoreType.DMA,
        pltpu.SemaphoreType.DMA,
        pltpu.SemaphoreType.DMA,
    ],
)(_sc_sort_body)


_LDJ_R, _LDJ_C = 8, 12544


def _ldj_body(o_ref):
    i0 = lax.broadcasted_iota(jnp.int32, (_LDJ_R, _LDJ_C), 0)
    i1 = lax.broadcasted_iota(jnp.int32, (_LDJ_R, _LDJ_C), 1)
    flat = i0 * _LDJ_C + i1
    val = jnp.log((flat + 1).astype(jnp.float32))
    s = jnp.sum(jnp.where(flat < _N, val, 0.0))
    o_ref[...] = jnp.full((_B, 1), -s, jnp.float32)


_ldj_call = pl.pallas_call(
    _ldj_body,
    out_shape=jax.ShapeDtypeStruct((_B, 1), jnp.float32),
)


def kernel(x):
    xb = lax.bitcast_convert_type(x, jnp.int32).reshape(_B * _N)
    z, _ = _sc_sort(xb)
    ldj = _ldj_call().reshape(_B)
    z = lax.bitcast_convert_type(z, jnp.float32).reshape(_B, _N)
    return (z, ldj)


# speedup vs baseline: 7.6127x; 1.0045x over previous
import functools

import numpy as np

import jax
import jax.numpy as jnp
from jax import lax
from jax.experimental import pallas as pl
from jax.experimental.pallas import tpu as pltpu
from jax.experimental.pallas import tpu_sc as plsc

_B = 64
_N = 100000
_W = 4000
_NWIN = _N // _W
_VPW = _W // 16
_U = 10
_NC = 2
_NS = 16
_ROWS_PER_TILE = _B // (_NC * _NS)

_PASSES = ((0, 2048), (11, 2048), (22, 1024))
_HIST_BASE = (0, 2048, 4096)
_HIST_SIZE = 5120
_OFFS_SIZE = 2048

_MSB = np.uint32(0x80000000)
_ALL1 = np.uint32(0xFFFFFFFF)


def _to_sortable(u):
    return jnp.where((u >> 31) == 1, u ^ _ALL1, u | _MSB)


def _from_sortable(k):
    return jnp.where((k >> 31) == 1, k ^ _MSB, k ^ _ALL1)


def _sc_sort_body(x_hbm, out_hbm, tmp_hbm, win_a, win_b, dst, hist, offs,
                  sem_a, sem_b, sem_t):
    cid = lax.axis_index("c")
    sid = lax.axis_index("s")
    wid = sid * _NC + cid
    zeros16 = jnp.zeros((16,), jnp.int32)
    ones16 = jnp.ones((16,), jnp.int32)

    def sweep(src, row_base, compute):
        def start(buf, sem, w):
            pltpu.async_copy(src.at[pl.ds(row_base + w * _W, _W)], buf, sem)

        def wait(buf, sem):
            pltpu.make_async_copy(
                src.at[pl.ds(row_base, _W)], buf, sem).wait()

        start(win_a, sem_a, 0)

        def pair(i, c):
            w0 = 2 * i
            wait(win_a, sem_a)
            start(win_b, sem_b, w0 + 1)
            compute(win_a, w0)
            wait(win_b, sem_b)
            start(win_a, sem_a, w0 + 2)
            compute(win_b, w0 + 1)
            return c

        lax.fori_loop(0, (_NWIN - 1) // 2, pair, 0)
        wait(win_a, sem_a)
        compute(win_a, _NWIN - 1)

    for r in range(_ROWS_PER_TILE):
        row = wid * _ROWS_PER_TILE + r
        row_base = row * _N

        def zero_body(i, c):
            hist[pl.ds(i * 16, 16)] = zeros16
            return c

        lax.fori_loop(0, _HIST_SIZE // 16, zero_body, 0)

        def hist_compute(buf, w):
            def body(g, c):
                for u in range(_U):
                    raw = buf[pl.ds((g * _U + u) * 16, 16)]
                    key = _to_sortable(plsc.bitcast(raw, jnp.uint32))
                    for (shift, bins), hb in zip(_PASSES, _HIST_BASE):
                        dig = plsc.bitcast(
                            (key >> shift) & (bins - 1), jnp.int32)
                        plsc.addupdate_scatter(hist, [dig + hb], ones16)
                return c

            lax.fori_loop(0, _VPW // _U, body, 0)

        sweep(x_hbm, row_base, hist_compute)

        for p, ((shift, bins), hb) in enumerate(zip(_PASSES, _HIST_BASE)):
            def scan_body(i, carry):
                h = hist[pl.ds(hb + i * 16, 16)]
                incl = plsc.cumsum(h)
                offs[pl.ds(i * 16, 16)] = incl - h + carry
                return carry + jnp.sum(h)

            lax.fori_loop(0, bins // 16, scan_body, jnp.int32(-1))

            if p > 0:
                pltpu.make_async_copy(
                    dst, tmp_hbm.at[pl.ds(row_base, _N)], sem_t).wait()

            last_pass = p == len(_PASSES) - 1

            def perm_compute(buf, w):
                def body(g, c):
                    for u in range(_U):
                        raw = buf[pl.ds((g * _U + u) * 16, 16)]
                        ku = plsc.bitcast(raw, jnp.uint32)
                        if p == 0:
                            ku = _to_sortable(ku)
                        dig = plsc.bitcast(
                            (ku >> shift) & (bins - 1), jnp.int32)
                        cnt, last = plsc.scan_count(dig)
                        base = plsc.load_gather(offs, [dig])
                        val = _from_sortable(ku) if last_pass else ku
                        plsc.store_scatter(
                            dst, [base + cnt], plsc.bitcast(val, jnp.int32))
                        plsc.addupdate_scatter(offs, [dig], cnt, mask=last)
                    return c

                lax.fori_loop(0, _VPW // _U, body, 0)

            src = x_hbm if p == 0 else tmp_hbm
            sweep(src, row_base, perm_compute)

            dst_hbm = out_hbm if last_pass else tmp_hbm
            pltpu.async_copy(dst, dst_hbm.at[pl.ds(row_base, _N)], sem_t)
            if last_pass:
                pltpu.make_async_copy(
                    dst, dst_hbm.at[pl.ds(row_base, _N)], sem_t).wait()


_sc_sort = functools.partial(
    pl.kernel,
    out_type=(
        jax.ShapeDtypeStruct((_B * _N,), jnp.int32),
        jax.ShapeDtypeStruct((_B * _N,), jnp.int32),
    ),
    mesh=plsc.VectorSubcoreMesh(
        core_axis_name="c", subcore_axis_name="s",
        num_cores=_NC, num_subcores=_NS),
    compiler_params=pltpu.CompilerParams(needs_layout_passes=False),
    scratch_types=[
        pltpu.VMEM((_W,), jnp.int32),
        pltpu.VMEM((_W,), jnp.int32),
        pltpu.VMEM((_N,), jnp.int32),
        pltpu.VMEM((_HIST_SIZE,), jnp.int32),
        pltpu.VMEM((_OFFS_SIZE,), jnp.int32),
        pltpu.SemaphoreType.DMA,
        pltpu.SemaphoreType.DMA,
        pltpu.SemaphoreType.DMA,
    ],
)(_sc_sort_body)


_LDJ_R, _LDJ_C = 8, 12544


def _ldj_body(o_ref):
    i0 = lax.broadcasted_iota(jnp.int32, (_LDJ_R, _LDJ_C), 0)
    i1 = lax.broadcasted_iota(jnp.int32, (_LDJ_R, _LDJ_C), 1)
    flat = i0 * _LDJ_C + i1
    val = jnp.log((flat + 1).astype(jnp.float32))
    s = jnp.sum(jnp.where(flat < _N, val, 0.0))
    o_ref[...] = jnp.full((_B, 1), -s, jnp.float32)


_ldj_call = pl.pallas_call(
    _ldj_body,
    out_shape=jax.ShapeDtypeStruct((_B, 1), jnp.float32),
)


def kernel(x):
    xb = lax.bitcast_convert_type(x, jnp.int32).reshape(_B * _N)
    z, _ = _sc_sort(xb)
    ldj = _ldj_call().reshape(_B)
    z = lax.bitcast_convert_type(z, jnp.float32).reshape(_B, _N)
    return (z, ldj)


# speedup vs baseline: 7.6220x; 1.0012x over previous
import functools

import numpy as np

import jax
import jax.numpy as jnp
from jax import lax
from jax.experimental import pallas as pl
from jax.experimental.pallas import tpu as pltpu
from jax.experimental.pallas import tpu_sc as plsc

_B = 64
_N = 100000
_W = 4000
_NWIN = _N // _W
_VPW = _W // 16
_U = 10
_NC = 2
_NS = 16
_ROWS_PER_TILE = _B // (_NC * _NS)

_PASSES = ((0, 2048), (11, 2048), (22, 1024))
_HIST_BASE = (0, 2048, 4096)
_HIST_SIZE = 5120
_OFFS_SIZE = 2048

_MSB = np.uint32(0x80000000)
_ALL1 = np.uint32(0xFFFFFFFF)


def _to_sortable(u):
    return jnp.where((u >> 31) == 1, u ^ _ALL1, u | _MSB)


def _from_sortable(k):
    return jnp.where((k >> 31) == 1, k ^ _MSB, k ^ _ALL1)


def _sc_sort_body(x_hbm, out_hbm, tmp_hbm, win_a, win_b, dst, hist, offs,
                  sem_a, sem_b, sem_t):
    cid = lax.axis_index("c")
    sid = lax.axis_index("s")
    wid = sid * _NC + cid
    zeros16 = jnp.zeros((16,), jnp.int32)
    ones16 = jnp.ones((16,), jnp.int32)

    def sweep(src, row_base, compute):
        def start(buf, sem, w):
            pltpu.async_copy(src.at[pl.ds(row_base + w * _W, _W)], buf, sem)

        def wait(buf, sem):
            pltpu.make_async_copy(
                src.at[pl.ds(row_base, _W)], buf, sem).wait()

        start(win_a, sem_a, 0)

        def pair(i, c):
            w0 = 2 * i
            wait(win_a, sem_a)
            start(win_b, sem_b, w0 + 1)
            compute(win_a, w0)
            wait(win_b, sem_b)
            start(win_a, sem_a, w0 + 2)
            compute(win_b, w0 + 1)
            return c

        lax.fori_loop(0, (_NWIN - 1) // 2, pair, 0)
        wait(win_a, sem_a)
        compute(win_a, _NWIN - 1)

    for r in range(_ROWS_PER_TILE):
        row = wid * _ROWS_PER_TILE + r
        row_base = row * _N

        def zero_body(i, c):
            hist[pl.ds(i * 16, 16)] = zeros16
            return c

        lax.fori_loop(0, _HIST_SIZE // 16, zero_body, 0)

        def hist_compute(buf, w):
            def body(g, c):
                for u in range(_U):
                    raw = buf[pl.ds((g * _U + u) * 16, 16)]
                    key = _to_sortable(plsc.bitcast(raw, jnp.uint32))
                    for (shift, bins), hb in zip(_PASSES, _HIST_BASE):
                        dig = plsc.bitcast(
                            (key >> shift) & (bins - 1), jnp.int32)
                        plsc.addupdate_scatter(hist, [dig + hb], ones16)
                return c

            lax.fori_loop(0, _VPW // _U, body, 0)

        sweep(x_hbm, row_base, hist_compute)

        for p, ((shift, bins), hb) in enumerate(zip(_PASSES, _HIST_BASE)):
            def scan_body(i, carry):
                h = hist[pl.ds(hb + i * 16, 16)]
                incl = plsc.cumsum(h)
                offs[pl.ds(i * 16, 16)] = incl - h + carry
                return carry + jnp.sum(h)

            lax.fori_loop(0, bins // 16, scan_body, jnp.int32(-1))

            if p > 0:
                pltpu.make_async_copy(
                    dst, tmp_hbm.at[pl.ds(row_base, _N)], sem_t).wait()

            last_pass = p == len(_PASSES) - 1

            def perm_compute(buf, w):
                def body(g, c):
                    for u in range(_U):
                        raw = buf[pl.ds((g * _U + u) * 16, 16)]
                        ku = plsc.bitcast(raw, jnp.uint32)
                        if p == 0:
                            ku = _to_sortable(ku)
                        dig = plsc.bitcast(
                            (ku >> shift) & (bins - 1), jnp.int32)
                        cnt, last = plsc.scan_count(dig)
                        base = plsc.load_gather(offs, [dig])
                        plsc.addupdate_scatter(offs, [dig], cnt, mask=last)
                        val = _from_sortable(ku) if last_pass else ku
                        plsc.store_scatter(
                            dst, [base + cnt], plsc.bitcast(val, jnp.int32))
                    return c

                lax.fori_loop(0, _VPW // _U, body, 0)

            src = x_hbm if p == 0 else tmp_hbm
            sweep(src, row_base, perm_compute)

            dst_hbm = out_hbm if last_pass else tmp_hbm
            pltpu.async_copy(dst, dst_hbm.at[pl.ds(row_base, _N)], sem_t)
            if last_pass:
                pltpu.make_async_copy(
                    dst, dst_hbm.at[pl.ds(row_base, _N)], sem_t).wait()


_sc_sort = functools.partial(
    pl.kernel,
    out_type=(
        jax.ShapeDtypeStruct((_B * _N,), jnp.int32),
        jax.ShapeDtypeStruct((_B * _N,), jnp.int32),
    ),
    mesh=plsc.VectorSubcoreMesh(
        core_axis_name="c", subcore_axis_name="s",
        num_cores=_NC, num_subcores=_NS),
    compiler_params=pltpu.CompilerParams(needs_layout_passes=False),
    scratch_types=[
        pltpu.VMEM((_W,), jnp.int32),
        pltpu.VMEM((_W,), jnp.int32),
        pltpu.VMEM((_N,), jnp.int32),
        pltpu.VMEM((_HIST_SIZE,), jnp.int32),
        pltpu.VMEM((_OFFS_SIZE,), jnp.int32),
        pltpu.SemaphoreType.DMA,
        pltpu.SemaphoreType.DMA,
        pltpu.SemaphoreType.DMA,
    ],
)(_sc_sort_body)


_LDJ_R, _LDJ_C = 8, 12544


def _ldj_body(o_ref):
    i0 = lax.broadcasted_iota(jnp.int32, (_LDJ_R, _LDJ_C), 0)
    i1 = lax.broadcasted_iota(jnp.int32, (_LDJ_R, _LDJ_C), 1)
    flat = i0 * _LDJ_C + i1
    val = jnp.log((flat + 1).astype(jnp.float32))
    s = jnp.sum(jnp.where(flat < _N, val, 0.0))
    o_ref[...] = jnp.full((_B, 1), -s, jnp.float32)


_ldj_call = pl.pallas_call(
    _ldj_body,
    out_shape=jax.ShapeDtypeStruct((_B, 1), jnp.float32),
)


def kernel(x):
    xb = lax.bitcast_convert_type(x, jnp.int32).reshape(_B * _N)
    z, _ = _sc_sort(xb)
    ldj = _ldj_call().reshape(_B)
    z = lax.bitcast_convert_type(z, jnp.float32).reshape(_B, _N)
    return (z, ldj)
